# Initial kernel scaffold; baseline (speedup 1.0000x reference)
import functools

import jax
import jax.numpy as jnp
from jax import lax
from jax.experimental import pallas as pl
from jax.experimental.pallas import tpu as pltpu
from jax.experimental.pallas import tpu_sc as plsc

NNODES = 100000
NEDGES = 1600000
NPAD = 102400
NP8 = NPAD // 8
NTILES = 32
SLICE = NPAD // 16
LANE_E = 128
KF = 8
NOUT = 49
ROWS_TILE = KF * NOUT
TOT_ROWS = NTILES * ROWS_TILE
EPAD = TOT_ROWS * LANE_E

_MESH = plsc.VectorSubcoreMesh(
    core_axis_name="c", subcore_axis_name="s", num_cores=2, num_subcores=16)


@functools.partial(
    pl.kernel,
    out_type=jax.ShapeDtypeStruct((2 * NPAD, 16), jnp.float32),
    mesh=_MESH,
    scratch_types=[
        pltpu.VMEM((KF, LANE_E), jnp.int32),
        pltpu.VMEM((LANE_E, 16), jnp.float32),
        pltpu.VMEM_SHARED((NPAD, 16), jnp.float32),
        pltpu.SemaphoreType.DMA,
    ],
)
def _sc_deg(dst_hbm, ones_hbm, zeros_hbm, out_hbm, dbuf, obuf, acc, ssem):
    c = lax.axis_index("c")
    s = lax.axis_index("s")
    wid = s * 2 + c
    r0 = s * SLICE
    pltpu.sync_copy(zeros_hbm.at[pl.ds(r0, SLICE)], acc.at[pl.ds(r0, SLICE)])
    pltpu.sync_copy(ones_hbm, obuf)
    plsc.subcore_barrier()
    base = wid * ROWS_TILE

    @pl.loop(0, NOUT)
    def _chunk(i):
        row0 = base + i * KF
        pltpu.sync_copy(dst_hbm.at[pl.ds(row0, KF)], dbuf)
        cps = [pltpu.async_copy(obuf, acc.at[dbuf.at[j]], ssem, add=True)
               for j in range(KF)]
        for cp in cps:
            cp.wait()

    plsc.subcore_barrier()
    pltpu.sync_copy(acc.at[pl.ds(r0, SLICE)],
                    out_hbm.at[pl.ds(c * NPAD + r0, SLICE)])


@functools.partial(
    pl.kernel,
    out_type=jax.ShapeDtypeStruct((2 * NPAD, 16), jnp.float32),
    mesh=_MESH,
    scratch_types=[
        pltpu.VMEM((KF, LANE_E), jnp.int32),
        pltpu.VMEM((KF, LANE_E), jnp.int32),
        pltpu.VMEM((KF, LANE_E, 16), jnp.float32),
        pltpu.VMEM_SHARED((NPAD, 16), jnp.float32),
        pltpu.SemaphoreType.DMA,
        pltpu.SemaphoreType.DMA,
    ],
)
def _sc_agg(src_hbm, dst_hbm, g_hbm, out_hbm, sbuf, dbuf, rbuf, acc,
            gsem, ssem):
    c = lax.axis_index("c")
    s = lax.axis_index("s")
    wid = s * 2 + c
    r0 = s * SLICE
    pltpu.sync_copy(g_hbm.at[pl.ds(r0, SLICE)], acc.at[pl.ds(r0, SLICE)])
    plsc.subcore_barrier()
    base = wid * ROWS_TILE

    @pl.loop(0, NOUT)
    def _chunk(i):
        row0 = base + i * KF
        pltpu.sync_copy(src_hbm.at[pl.ds(row0, KF)], sbuf)
        pltpu.sync_copy(dst_hbm.at[pl.ds(row0, KF)], dbuf)
        gcp = [pltpu.async_copy(g_hbm.at[sbuf.at[j]], rbuf.at[j], gsem)
               for j in range(KF)]
        for cp in gcp:
            cp.wait()
        scp = [pltpu.async_copy(rbuf.at[j], acc.at[dbuf.at[j]], ssem,
                                add=True)
               for j in range(KF)]
        for cp in scp:
            cp.wait()

    plsc.subcore_barrier()
    pltpu.sync_copy(acc.at[pl.ds(r0, SLICE)],
                    out_hbm.at[pl.ds(c * NPAD + r0, SLICE)])


_BR = 1600


def _tc_prep_body(d0, d1, xp, dinv_out, g1_out):
    dinv = lax.rsqrt(d0[...] + d1[...] + 1.0)
    dinv_out[...] = dinv
    g1_out[...] = dinv * xp[...]


def _tc_layer_body(wblk, brep, y0, y1, g, dinv, gnext_out):
    z = dinv[...] * (y0[...] + y1[...] - g[...])
    h = jnp.tanh(
        jnp.dot(z, wblk[...], preferred_element_type=jnp.float32) + brep[...])
    gnext_out[...] = dinv[...] * h


_BR2 = 4096


def _tc_final_body(w3, b3, wc, bc2, y0, y1, g, dinv, h_out, o_out):
    z = dinv[...] * (y0[...] + y1[...] - g[...])
    h = jnp.tanh(
        jnp.dot(z, w3[...], preferred_element_type=jnp.float32) + b3[...])
    h_out[...] = h
    o_out[...] = (
        jnp.dot(h, wc[...], preferred_element_type=jnp.float32) + bc2[...])


def _row_spec(br, ncols, offset_blocks=0):
    return pl.BlockSpec((br, ncols), lambda i, o=offset_blocks: (i + o, 0))


def _fixed_spec(shape):
    return pl.BlockSpec(shape, lambda i: (0, 0))


def kernel(x, edge_index, W1, b1, Wh0, bh0, Wh1, bh1, W3, b3, Wc, bc):
    f32 = jnp.float32
    src = edge_index[0]
    dst = edge_index[1]
    pad = EPAD - NEDGES
    padv = jnp.full((pad,), NNODES, jnp.int32)
    src2d = jnp.concatenate([src, padv]).reshape(TOT_ROWS, LANE_E)
    dst2d = jnp.concatenate([dst, padv]).reshape(TOT_ROWS, LANE_E)
    xpad = jnp.zeros((NPAD, 16), f32).at[:NNODES, :3].set(x)
    zeros16 = jnp.zeros((NPAD, 16), f32)
    ones_payload = jnp.ones((LANE_E, 16), f32)

    def pad16(w, b):
        wp = jnp.zeros((16, 16), f32).at[:w.shape[0], :w.shape[1]].set(w)
        bp = jnp.zeros((16,), f32).at[:b.shape[0]].set(b)
        return (jnp.kron(jnp.eye(8, dtype=f32), wp),
                jnp.tile(bp, 8).reshape(1, 128))

    wblk1, brep1 = pad16(W1, b1)
    wblkh0, breph0 = pad16(Wh0, bh0)
    wblkh1, breph1 = pad16(Wh1, bh1)
    w3p = jnp.zeros((16, 128), f32).at[:15, :120].set(W3)
    b3p = jnp.zeros((1, 128), f32).at[0, :120].set(b3)
    wcp = jnp.zeros((128, 8), f32).at[:120, :2].set(Wc)
    bcp = jnp.zeros((1, 8), f32).at[0, :2].set(bc)

    degp = _sc_deg(dst2d, ones_payload, zeros16)
    degp128 = degp.reshape(2 * NP8, 128)

    nblk = NP8 // _BR
    dinv128, g128 = pl.pallas_call(
        _tc_prep_body,
        grid=(nblk,),
        in_specs=[_row_spec(_BR, 128), _row_spec(_BR, 128, nblk),
                  _row_spec(_BR, 128)],
        out_specs=[_row_spec(_BR, 128), _row_spec(_BR, 128)],
        out_shape=[jax.ShapeDtypeStruct((NP8, 128), f32),
                   jax.ShapeDtypeStruct((NP8, 128), f32)],
    )(degp128, degp128, xpad.reshape(NP8, 128))

    for wblk, brep in ((wblk1, brep1), (wblkh0, breph0), (wblkh1, breph1)):
        y = _sc_agg(src2d, dst2d, g128.reshape(NPAD, 16))
        y128 = y.reshape(2 * NP8, 128)
        g128 = pl.pallas_call(
            _tc_layer_body,
            grid=(nblk,),
            in_specs=[_fixed_spec((128, 128)), _fixed_spec((1, 128)),
                      _row_spec(_BR, 128), _row_spec(_BR, 128, nblk),
                      _row_spec(_BR, 128), _row_spec(_BR, 128)],
            out_specs=_row_spec(_BR, 128),
            out_shape=jax.ShapeDtypeStruct((NP8, 128), f32),
        )(wblk, brep, y128, y128, g128, dinv128)

    y = _sc_agg(src2d, dst2d, g128.reshape(NPAD, 16))
    nblk2 = NPAD // _BR2
    hfull, ofull = pl.pallas_call(
        _tc_final_body,
        grid=(nblk2,),
        in_specs=[_fixed_spec((16, 128)), _fixed_spec((1, 128)),
                  _fixed_spec((128, 8)), _fixed_spec((1, 8)),
                  _row_spec(_BR2, 16), _row_spec(_BR2, 16, nblk2),
                  _row_spec(_BR2, 16), _row_spec(_BR2, 16)],
        out_specs=[_row_spec(_BR2, 128), _row_spec(_BR2, 8)],
        out_shape=[jax.ShapeDtypeStruct((NPAD, 128), f32),
                   jax.ShapeDtypeStruct((NPAD, 8), f32)],
    )(w3p, b3p, wcp, bcp, y, y, g128.reshape(NPAD, 16),
      dinv128.reshape(NPAD, 16))

    out = ofull[:NNODES, :2]
    h = hfull[:NNODES, :120]
    return (out, h)


# speedup vs baseline: 39.2707x; 39.2707x over previous
import functools

import jax
import jax.numpy as jnp
from jax import lax
from jax.experimental import pallas as pl
from jax.experimental.pallas import tpu as pltpu
from jax.experimental.pallas import tpu_sc as plsc

NNODES = 100000
NEDGES = 1600000
NPAD = 102400
NP8 = NPAD // 8
NTILES = 32
SLICE = NPAD // 16
LANE_E = 128
KF = 8
NOUT = 49
ROWS_TILE = KF * NOUT
TOT_ROWS = NTILES * ROWS_TILE
EPAD = TOT_ROWS * LANE_E

_MESH = plsc.VectorSubcoreMesh(
    core_axis_name="c", subcore_axis_name="s", num_cores=2, num_subcores=16)

_SC_PARAMS = pltpu.CompilerParams(use_tc_tiling_on_sc=False)


@functools.partial(
    pl.kernel,
    out_type=jax.ShapeDtypeStruct((2 * NPAD, 16), jnp.float32),
    mesh=_MESH,
    scratch_types=[
        pltpu.VMEM((KF, LANE_E), jnp.int32),
        pltpu.VMEM((LANE_E, 16), jnp.float32),
        pltpu.VMEM_SHARED((NPAD, 16), jnp.float32),
        pltpu.SemaphoreType.DMA,
    ],
    compiler_params=_SC_PARAMS,
)
def _sc_deg(dst_hbm, ones_hbm, zeros_hbm, out_hbm, dbuf, obuf, acc, ssem):
    c = lax.axis_index("c")
    s = lax.axis_index("s")
    wid = s * 2 + c
    r0 = s * SLICE
    pltpu.sync_copy(zeros_hbm.at[pl.ds(r0, SLICE)], acc.at[pl.ds(r0, SLICE)])
    pltpu.sync_copy(ones_hbm, obuf)
    plsc.subcore_barrier()
    base = wid * ROWS_TILE

    @pl.loop(0, NOUT)
    def _chunk(i):
        row0 = base + i * KF
        pltpu.sync_copy(dst_hbm.at[pl.ds(row0, KF)], dbuf)
        cps = [pltpu.async_copy(obuf, acc.at[dbuf.at[j]], ssem, add=True)
               for j in range(KF)]
        for cp in cps:
            cp.wait()

    plsc.subcore_barrier()
    pltpu.sync_copy(acc.at[pl.ds(r0, SLICE)],
                    out_hbm.at[pl.ds(c * NPAD + r0, SLICE)])


@functools.partial(
    pl.kernel,
    out_type=jax.ShapeDtypeStruct((2 * NPAD, 16), jnp.float32),
    mesh=_MESH,
    scratch_types=[
        pltpu.VMEM((KF, LANE_E), jnp.int32),
        pltpu.VMEM((KF, LANE_E), jnp.int32),
        pltpu.VMEM((KF, LANE_E, 16), jnp.float32),
        pltpu.VMEM_SHARED((NPAD, 16), jnp.float32),
        pltpu.SemaphoreType.DMA,
        pltpu.SemaphoreType.DMA,
    ],
    compiler_params=_SC_PARAMS,
)
def _sc_agg(src_hbm, dst_hbm, g_hbm, out_hbm, sbuf, dbuf, rbuf, acc,
            gsem, ssem):
    c = lax.axis_index("c")
    s = lax.axis_index("s")
    wid = s * 2 + c
    r0 = s * SLICE
    pltpu.sync_copy(g_hbm.at[pl.ds(r0, SLICE)], acc.at[pl.ds(r0, SLICE)])
    plsc.subcore_barrier()
    base = wid * ROWS_TILE

    @pl.loop(0, NOUT)
    def _chunk(i):
        row0 = base + i * KF
        pltpu.sync_copy(src_hbm.at[pl.ds(row0, KF)], sbuf)
        pltpu.sync_copy(dst_hbm.at[pl.ds(row0, KF)], dbuf)
        gcp = [pltpu.async_copy(g_hbm.at[sbuf.at[j]], rbuf.at[j], gsem)
               for j in range(KF)]
        for cp in gcp:
            cp.wait()
        scp = [pltpu.async_copy(rbuf.at[j], acc.at[dbuf.at[j]], ssem,
                                add=True)
               for j in range(KF)]
        for cp in scp:
            cp.wait()

    plsc.subcore_barrier()
    pltpu.sync_copy(acc.at[pl.ds(r0, SLICE)],
                    out_hbm.at[pl.ds(c * NPAD + r0, SLICE)])


_BR = 1600


def _tc_prep_body(d0, d1, xp, dinv_out, g1_out):
    dinv = 1.0 / jnp.sqrt(d0[...] + d1[...] + 1.0)
    dinv_out[...] = dinv
    g1_out[...] = dinv * xp[...]


def _tc_layer_body(wblk, brep, y0, y1, g, dinv, gnext_out):
    z = dinv[...] * (y0[...] + y1[...] - g[...])
    h = jnp.tanh(
        jnp.dot(z, wblk[...], preferred_element_type=jnp.float32,
                precision=lax.Precision.HIGHEST) + brep[...])
    gnext_out[...] = dinv[...] * h


_BR2 = 4096


def _tc_final_body(w3, b3, wc, bc2, y0, y1, g, dinv, h_out, o_out):
    z = dinv[...] * (y0[...] + y1[...] - g[...])
    h = jnp.tanh(
        jnp.dot(z, w3[...], preferred_element_type=jnp.float32,
                precision=lax.Precision.HIGHEST) + b3[...])
    h_out[...] = h
    o_out[...] = (
        jnp.dot(h, wc[...], preferred_element_type=jnp.float32,
                precision=lax.Precision.HIGHEST) + bc2[...])


def _row_spec(br, ncols, offset_blocks=0):
    return pl.BlockSpec((br, ncols), lambda i, o=offset_blocks: (i + o, 0))


def _fixed_spec(shape):
    return pl.BlockSpec(shape, lambda i: (0, 0))


def kernel(x, edge_index, W1, b1, Wh0, bh0, Wh1, bh1, W3, b3, Wc, bc):
    f32 = jnp.float32
    src = edge_index[0]
    dst = edge_index[1]
    pad = EPAD - NEDGES
    padv = jnp.full((pad,), NNODES, jnp.int32)
    src2d = jnp.concatenate([src, padv]).reshape(TOT_ROWS, LANE_E)
    dst2d = jnp.concatenate([dst, padv]).reshape(TOT_ROWS, LANE_E)
    xpad = jnp.zeros((NPAD, 16), f32).at[:NNODES, :3].set(x)
    zeros16 = jnp.zeros((NPAD, 16), f32)
    ones_payload = jnp.ones((LANE_E, 16), f32)

    def pad16(w, b):
        wp = jnp.zeros((16, 16), f32).at[:w.shape[0], :w.shape[1]].set(w)
        bp = jnp.zeros((16,), f32).at[:b.shape[0]].set(b)
        return (jnp.kron(jnp.eye(8, dtype=f32), wp),
                jnp.tile(bp, 8).reshape(1, 128))

    wblk1, brep1 = pad16(W1, b1)
    wblkh0, breph0 = pad16(Wh0, bh0)
    wblkh1, breph1 = pad16(Wh1, bh1)
    w3p = jnp.zeros((16, 128), f32).at[:15, :120].set(W3)
    b3p = jnp.zeros((1, 128), f32).at[0, :120].set(b3)
    wcp = jnp.zeros((128, 8), f32).at[:120, :2].set(Wc)
    bcp = jnp.zeros((1, 8), f32).at[0, :2].set(bc)

    degp = _sc_deg(dst2d, ones_payload, zeros16)
    degp128 = degp.reshape(2 * NP8, 128)

    nblk = NP8 // _BR
    dinv128, g128 = pl.pallas_call(
        _tc_prep_body,
        grid=(nblk,),
        in_specs=[_row_spec(_BR, 128), _row_spec(_BR, 128, nblk),
                  _row_spec(_BR, 128)],
        out_specs=[_row_spec(_BR, 128), _row_spec(_BR, 128)],
        out_shape=[jax.ShapeDtypeStruct((NP8, 128), f32),
                   jax.ShapeDtypeStruct((NP8, 128), f32)],
    )(degp128, degp128, xpad.reshape(NP8, 128))

    for wblk, brep in ((wblk1, brep1), (wblkh0, breph0), (wblkh1, breph1)):
        y = _sc_agg(src2d, dst2d, g128.reshape(NPAD, 16))
        y128 = y.reshape(2 * NP8, 128)
        g128 = pl.pallas_call(
            _tc_layer_body,
            grid=(nblk,),
            in_specs=[_fixed_spec((128, 128)), _fixed_spec((1, 128)),
                      _row_spec(_BR, 128), _row_spec(_BR, 128, nblk),
                      _row_spec(_BR, 128), _row_spec(_BR, 128)],
            out_specs=_row_spec(_BR, 128),
            out_shape=jax.ShapeDtypeStruct((NP8, 128), f32),
        )(wblk, brep, y128, y128, g128, dinv128)

    y = _sc_agg(src2d, dst2d, g128.reshape(NPAD, 16))
    nblk2 = NPAD // _BR2
    hfull, ofull = pl.pallas_call(
        _tc_final_body,
        grid=(nblk2,),
        in_specs=[_fixed_spec((16, 128)), _fixed_spec((1, 128)),
                  _fixed_spec((128, 8)), _fixed_spec((1, 8)),
                  _row_spec(_BR2, 16), _row_spec(_BR2, 16, nblk2),
                  _row_spec(_BR2, 16), _row_spec(_BR2, 16)],
        out_specs=[_row_spec(_BR2, 128), _row_spec(_BR2, 8)],
        out_shape=[jax.ShapeDtypeStruct((NPAD, 128), f32),
                   jax.ShapeDtypeStruct((NPAD, 8), f32)],
    )(w3p, b3p, wcp, bcp, y, y, g128.reshape(NPAD, 16),
      dinv128.reshape(NPAD, 16))

    out = ofull[:NNODES, :2]
    h = hfull[:NNODES, :120]
    return (out, h)
